# speedup vs baseline: 1.0405x; 1.0405x over previous
import functools

import jax
import jax.numpy as jnp
from jax import lax
from jax.experimental import pallas as pl
from jax.experimental.pallas import tpu as pltpu
from jax.experimental.pallas import tpu_sc as plsc

_BN = 512
_BK = 2048


def _dist_argmin_body(zsq_ref, esq_ref, z_ref, e_ref, idx_ref, d2_ref):
    bn = z_ref.shape[0]
    k_total = e_ref.shape[0]
    zb = z_ref[...]
    zsq = zsq_ref[0, 0, :][:, None]
    run_min = jnp.full((bn, 1), jnp.inf, dtype=jnp.float32)
    run_idx = jnp.zeros((bn, 1), dtype=jnp.int32)
    for k in range(k_total // _BK):
        e_chunk = e_ref[k * _BK:(k + 1) * _BK, :]
        esq = esq_ref[0, 0, k * _BK:(k + 1) * _BK][None, :]
        scores = lax.dot_general(
            zb, e_chunk, (((1,), (1,)), ((), ())),
            preferred_element_type=jnp.float32)
        d2 = (zsq + esq) - 2.0 * scores
        dist = jnp.sqrt(jnp.maximum(d2, 0.0))
        cmin = jnp.min(dist, axis=1, keepdims=True)
        ciota = lax.broadcasted_iota(jnp.int32, (bn, _BK), 1)
        cidx = jnp.min(jnp.where(dist == cmin, ciota, k_total),
                       axis=1, keepdims=True) + k * _BK
        upd = cmin < run_min
        run_idx = jnp.where(upd, cidx, run_idx)
        run_min = jnp.where(upd, cmin, run_min)
    idx_ref[0, 0, :] = run_idx[:, 0]
    d2_ref[0, 0, :] = (run_min * run_min)[:, 0]


def _dist_argmin(z_flat, z_sq, e_sq, embeddings):
    n, c = z_flat.shape
    k_total = embeddings.shape[0]
    nb = n // _BN
    zsq3 = z_sq.reshape(nb, 1, _BN)
    esq3 = e_sq.reshape(1, 1, k_total)
    idx, d2 = pl.pallas_call(
        _dist_argmin_body,
        grid=(nb,),
        in_specs=[
            pl.BlockSpec((1, 1, _BN), lambda i: (i, 0, 0)),
            pl.BlockSpec((1, 1, k_total), lambda i: (0, 0, 0)),
            pl.BlockSpec((_BN, c), lambda i: (i, 0)),
            pl.BlockSpec((k_total, c), lambda i: (0, 0)),
        ],
        out_specs=[
            pl.BlockSpec((1, 1, _BN), lambda i: (i, 0, 0)),
            pl.BlockSpec((1, 1, _BN), lambda i: (i, 0, 0)),
        ],
        out_shape=[
            jax.ShapeDtypeStruct((nb, 1, _BN), jnp.int32),
            jax.ShapeDtypeStruct((nb, 1, _BN), jnp.float32),
        ],
        compiler_params=pltpu.CompilerParams(
            dimension_semantics=("parallel",)),
    )(zsq3, esq3, z_flat, embeddings)
    return idx.reshape(n), d2.reshape(n)


def _sc_gather(embeddings, idx):
    n = idx.shape[0]
    c = embeddings.shape[1]
    info = plsc.get_sparse_core_info()
    nw = info.num_cores * info.num_subcores
    b_per_w = n // nw
    mesh = plsc.VectorSubcoreMesh(core_axis_name="c", subcore_axis_name="s")

    @functools.partial(
        pl.kernel, mesh=mesh,
        out_type=jax.ShapeDtypeStruct((n, c), jnp.float32),
        scratch_types=[
            pltpu.VMEM((b_per_w,), jnp.int32),
            pltpu.VMEM((b_per_w, c), jnp.float32),
            pltpu.SemaphoreType.DMA,
        ],
    )
    def gather_kernel(table_hbm, idx_hbm, out_hbm, idx_v, rows_v, sem):
        wid = lax.axis_index("s") * info.num_cores + lax.axis_index("c")
        base = wid * b_per_w
        pltpu.sync_copy(idx_hbm.at[pl.ds(base, b_per_w)], idx_v)
        pltpu.async_copy(table_hbm.at[idx_v], rows_v, sem).wait()
        pltpu.sync_copy(rows_v, out_hbm.at[pl.ds(base, b_per_w)])

    return gather_kernel(embeddings, idx)


def kernel(z, embeddings):
    bs, h, w, d, c = z.shape
    z_flat = z.reshape(-1, c)
    z_sq = jnp.sum(z_flat ** 2, axis=1, keepdims=True)
    e_sq = jnp.sum(embeddings ** 2, axis=1)
    idx, d2min = _dist_argmin(z_flat, z_sq, e_sq, embeddings)
    q_flat = _sc_gather(embeddings, idx)
    quantized = q_flat.reshape(bs, h, w, d, c)
    loss = jnp.sum(d2min) / z.size
    quantized_st = z + lax.stop_gradient(quantized - z)
    encoding_indices = idx.reshape(bs, h, w, d)
    return (quantized_st, loss, loss, encoding_indices)


# speedup vs baseline: 1.2432x; 1.1949x over previous
import functools

import jax
import jax.numpy as jnp
from jax import lax
from jax.experimental import pallas as pl
from jax.experimental.pallas import tpu as pltpu
from jax.experimental.pallas import tpu_sc as plsc

_BN = 256
_BK = 256


def _dist_argmin_body(zsq_ref, esq_ref, z_ref, e_ref, idx_ref, d2_ref):
    bn = z_ref.shape[0]
    k_total = e_ref.shape[0]
    nk = k_total // _BK
    zb = z_ref[...]
    zsq = zsq_ref[0, 0, :][:, None]
    run_min = jnp.full((bn, _BK), jnp.inf, dtype=jnp.float32)
    run_k = jnp.zeros((bn, _BK), dtype=jnp.int32)
    for k in range(nk):
        e_chunk = e_ref[k * _BK:(k + 1) * _BK, :]
        esq = esq_ref[0, 0, k * _BK:(k + 1) * _BK][None, :]
        scores = lax.dot_general(
            zb, e_chunk, (((1,), (1,)), ((), ())),
            preferred_element_type=jnp.float32)
        d2 = (zsq + esq) - 2.0 * scores
        c = jnp.maximum(d2, 0.0)
        dist = c * lax.rsqrt(c)
        dist = jnp.where(c == 0.0, 0.0, dist)
        better = dist < run_min
        run_k = jnp.where(better, k, run_k)
        run_min = jnp.minimum(run_min, dist)
    lane = lax.broadcasted_iota(jnp.int32, (bn, _BK), 1)
    gidx = run_k * _BK + lane
    m = jnp.min(run_min, axis=1, keepdims=True)
    idx = jnp.min(jnp.where(run_min == m, gidx, k_total),
                  axis=1, keepdims=True)
    idx_ref[0, 0, :] = idx[:, 0]
    d2_ref[0, 0, :] = (m * m)[:, 0]


def _dist_argmin(z_flat, z_sq, e_sq, embeddings):
    n, c = z_flat.shape
    k_total = embeddings.shape[0]
    nb = n // _BN
    zsq3 = z_sq.reshape(nb, 1, _BN)
    esq3 = e_sq.reshape(1, 1, k_total)
    idx, d2 = pl.pallas_call(
        _dist_argmin_body,
        grid=(nb,),
        in_specs=[
            pl.BlockSpec((1, 1, _BN), lambda i: (i, 0, 0)),
            pl.BlockSpec((1, 1, k_total), lambda i: (0, 0, 0)),
            pl.BlockSpec((_BN, c), lambda i: (i, 0)),
            pl.BlockSpec((k_total, c), lambda i: (0, 0)),
        ],
        out_specs=[
            pl.BlockSpec((1, 1, _BN), lambda i: (i, 0, 0)),
            pl.BlockSpec((1, 1, _BN), lambda i: (i, 0, 0)),
        ],
        out_shape=[
            jax.ShapeDtypeStruct((nb, 1, _BN), jnp.int32),
            jax.ShapeDtypeStruct((nb, 1, _BN), jnp.float32),
        ],
        compiler_params=pltpu.CompilerParams(
            dimension_semantics=("parallel",)),
    )(zsq3, esq3, z_flat, embeddings)
    return idx.reshape(n), d2.reshape(n)


def _sc_gather(embeddings, idx):
    n = idx.shape[0]
    c = embeddings.shape[1]
    info = plsc.get_sparse_core_info()
    nw = info.num_cores * info.num_subcores
    b_per_w = n // nw
    mesh = plsc.VectorSubcoreMesh(core_axis_name="c", subcore_axis_name="s")

    @functools.partial(
        pl.kernel, mesh=mesh,
        out_type=jax.ShapeDtypeStruct((n, c), jnp.float32),
        scratch_types=[
            pltpu.VMEM((b_per_w,), jnp.int32),
            pltpu.VMEM((b_per_w, c), jnp.float32),
            pltpu.SemaphoreType.DMA,
        ],
    )
    def gather_kernel(table_hbm, idx_hbm, out_hbm, idx_v, rows_v, sem):
        wid = lax.axis_index("s") * info.num_cores + lax.axis_index("c")
        base = wid * b_per_w
        pltpu.sync_copy(idx_hbm.at[pl.ds(base, b_per_w)], idx_v)
        pltpu.async_copy(table_hbm.at[idx_v], rows_v, sem).wait()
        pltpu.sync_copy(rows_v, out_hbm.at[pl.ds(base, b_per_w)])

    return gather_kernel(embeddings, idx)


def kernel(z, embeddings):
    bs, h, w, d, c = z.shape
    z_flat = z.reshape(-1, c)
    z_sq = jnp.sum(z_flat ** 2, axis=1, keepdims=True)
    e_sq = jnp.sum(embeddings ** 2, axis=1)
    idx, d2min = _dist_argmin(z_flat, z_sq, e_sq, embeddings)
    q_flat = _sc_gather(embeddings, idx)
    quantized = q_flat.reshape(bs, h, w, d, c)
    loss = jnp.sum(d2min) / z.size
    quantized_st = z + lax.stop_gradient(quantized - z)
    encoding_indices = idx.reshape(bs, h, w, d)
    return (quantized_st, loss, loss, encoding_indices)
